# baseline (speedup 1.0000x reference)
import dataclasses
import functools

import jax
import jax.numpy as jnp
from jax import lax
from jax.experimental import pallas as pl
from jax.experimental.pallas import tpu as pltpu
from jax.experimental.pallas import tpu_sc as plsc

N_USERS = 5000
N_ITEMS = 5000
D = 256
E = 160000
NC = 2
NT = 16
LANES = 16
SCAN = 1024
NCHUNK = 160
EPAD = SCAN * NCHUNK
ROWS = 5120
RPT = ROWS // NT
ACCR = RPT + 8
DW = D // 2
GB = 128
HB = GB // 2
PCAP = 2048
DUMMY = ROWS - 1


def _sc_accumulate(x, pidx):
    mesh = plsc.VectorSubcoreMesh(core_axis_name="c", subcore_axis_name="s")
    cp = pltpu.CompilerParams()
    if "needs_layout_passes" in pltpu.CompilerParams.__dataclass_fields__:
        cp = dataclasses.replace(cp, needs_layout_passes=False)

    @functools.partial(
        pl.kernel,
        mesh=mesh,
        compiler_params=cp,
        out_type=(
            jax.ShapeDtypeStruct((NC, ROWS, DW), jnp.float32),
            jax.ShapeDtypeStruct((NC, ROWS, DW), jnp.float32),
            jax.ShapeDtypeStruct((NC * ROWS,), jnp.float32),
        ),
        scratch_types=[
            pltpu.VMEM((ACCR, DW), jnp.float32),
            pltpu.VMEM((ACCR, DW), jnp.float32),
            pltpu.VMEM((HB, DW), jnp.int32),
            pltpu.VMEM((HB, DW), jnp.int32),
            pltpu.VMEM((2, SCAN), jnp.int32),
            pltpu.VMEM((PCAP,), jnp.int32),
            pltpu.VMEM((PCAP,), jnp.int32),
            pltpu.VMEM((ACCR,), jnp.float32),
            pltpu.SMEM((8,), jnp.int32),
            pltpu.SemaphoreType.DMA,
            pltpu.SemaphoreType.DMA,
            pltpu.SemaphoreType.DMA,
        ],
    )
    def k(x_hbm, pidx_hbm, sums_e_hbm, sums_o_hbm, cnt_hbm,
          acc_e, acc_o, rows0, rows1, pc, pg, pd, cnt, cur_ref,
          semg0, semg1, semi):
        c = lax.axis_index("c")
        s = lax.axis_index("s")
        lo = s * RPT

        @pl.loop(0, ACCR)
        def _(r):
            for j in range(0, DW, LANES):
                acc_e.at[r, pl.ds(j, LANES)][...] = jnp.zeros(
                    (LANES,), jnp.float32)
                acc_o.at[r, pl.ds(j, LANES)][...] = jnp.zeros(
                    (LANES,), jnp.float32)

        @pl.loop(0, ACCR, step=LANES)
        def _(r):
            cnt.at[pl.ds(r, LANES)][...] = jnp.zeros((LANES,), jnp.float32)

        cur_ref[0] = 0

        def gather_half(base, rbuf, sg, start):
            h = pltpu.make_async_copy(
                x_hbm.at[pg.at[pl.ds(base, HB)]], rbuf, sg)
            if start:
                h.start()
            else:
                h.wait()

        hi_mask = jnp.int32(-65536)

        def acc_half(rbuf, pdbase):
            @pl.loop(0, HB, step=LANES)
            def _(e):
                d_v = pd.at[pl.ds(pdbase + e, LANES)][...]
                plsc.addupdate_scatter(cnt, [d_v], ones_f)
                dd = [d_v[li] for li in range(LANES)]

                def loads(li):
                    return [rbuf.at[e + li, pl.ds(j, LANES)][...]
                            for j in range(0, DW, LANES)]

                vals = loads(0)
                for li in range(LANES):
                    cur_vals = vals
                    if li + 1 < LANES:
                        vals = loads(li + 1)
                    for jq, j in enumerate(range(0, DW, LANES)):
                        w_v = cur_vals[jq]
                        ev = plsc.bitcast(w_v << 16, jnp.float32)
                        od = plsc.bitcast(w_v & hi_mask, jnp.float32)
                        plsc.addupdate(acc_e.at[dd[li], pl.ds(j, LANES)], ev)
                        plsc.addupdate(acc_o.at[dd[li], pl.ds(j, LANES)], od)

        def drain_full_batches():
            cur = cur_ref[0]
            nb = cur // GB

            @pl.when(nb > 0)
            def _():
                gather_half(0, rows0, semg0, start=True)
                gather_half(HB, rows1, semg1, start=True)

                def body(q, carry):
                    base = q * GB
                    gather_half(base, rows0, semg0, start=False)
                    acc_half(rows0, base)

                    @pl.when(q + 1 < nb)
                    def _():
                        gather_half(base + GB, rows0, semg0, start=True)

                    gather_half(base + HB, rows1, semg1, start=False)
                    acc_half(rows1, base + HB)

                    @pl.when(q + 1 < nb)
                    def _():
                        gather_half(base + GB + HB, rows1, semg1, start=True)

                    return carry

                lax.fori_loop(0, nb, body, 0)

                rbase = nb * GB
                for j in range(0, GB, LANES):
                    pg.at[pl.ds(j, LANES)][...] = (
                        pg.at[pl.ds(rbase + j, LANES)][...])
                    pd.at[pl.ds(j, LANES)][...] = (
                        pd.at[pl.ds(rbase + j, LANES)][...])
                cur_ref[0] = cur - nb * GB

        ones_f = jnp.ones((LANES,), jnp.float32)

        def idx_copy(chunk, b, start):
            h = pltpu.make_async_copy(
                pidx_hbm.at[c, pl.ds(chunk * SCAN, SCAN)], pc.at[b], semi)
            if start:
                h.start()
            else:
                h.wait()

        def chunk_body(ch, b):
            idx_copy(ch, b, start=False)

            def grp(gi, cur):
                base = gi * LANES
                w_v = pc.at[b, pl.ds(base, LANES)][...]
                g_v = lax.shift_right_logical(w_v, 16)
                d_v = (w_v & 0xFFFF) - lo
                m = plsc.bitcast(d_v, jnp.uint32) < jnp.uint32(RPT)
                plsc.store_compressed(pg.at[pl.ds(cur, LANES)], g_v, mask=m)
                plsc.store_compressed(pd.at[pl.ds(cur, LANES)], d_v, mask=m)
                return cur + plsc.all_reduce_population_count(m)[0]

            cur_ref[0] = lax.fori_loop(0, SCAN // LANES, grp, cur_ref[0])
            idx_copy(ch + 2, b, start=True)
            drain_full_batches()

        idx_copy(0, 0, start=True)
        idx_copy(1, 1, start=True)

        @pl.loop(0, NCHUNK, step=2)
        def _(ch):
            chunk_body(ch, 0)
            chunk_body(ch + 1, 1)

        idx_copy(NCHUNK, 0, start=False)
        idx_copy(NCHUNK + 1, 1, start=False)

        cur = cur_ref[0]
        for j in range(0, GB, LANES):
            pg.at[pl.ds(cur + j, LANES)][...] = jnp.zeros((LANES,), jnp.int32)
            pd.at[pl.ds(cur + j, LANES)][...] = jnp.full(
                (LANES,), RPT, jnp.int32)
        cur_ref[0] = cur + GB
        drain_full_batches()

        pltpu.sync_copy(acc_e.at[pl.ds(0, RPT)],
                        sums_e_hbm.at[c, pl.ds(lo, RPT)])
        pltpu.sync_copy(acc_o.at[pl.ds(0, RPT)],
                        sums_o_hbm.at[c, pl.ds(lo, RPT)])
        pltpu.sync_copy(cnt.at[pl.ds(0, RPT)],
                        cnt_hbm.at[pl.ds(c * ROWS + lo, RPT)])

    return k(x, pidx)


_BR = 200


def _scale_body(sums_e_ref, sums_o_ref, cnt_ref, out_ref):
    w = 1.0 / jnp.maximum(cnt_ref[0], 1.0)
    out_ref[...] = jnp.concatenate(
        [sums_e_ref[0] * w, sums_o_ref[0] * w], axis=1)


def _scale(sums_e, sums_o, cnt):
    nb = N_USERS // _BR
    return pl.pallas_call(
        _scale_body,
        grid=(NC, nb),
        in_specs=[
            pl.BlockSpec((1, _BR, DW), lambda c, b: (c, b, 0)),
            pl.BlockSpec((1, _BR, DW), lambda c, b: (c, b, 0)),
            pl.BlockSpec((1, _BR, 1), lambda c, b: (c, b, 0)),
        ],
        out_specs=pl.BlockSpec((_BR, D), lambda c, b: (c * nb + b, 0)),
        out_shape=jax.ShapeDtypeStruct((N_USERS + N_ITEMS, D), jnp.float32),
    )(sums_e, sums_o, cnt)


def kernel(x, u, i):
    pad = EPAD + 2 * SCAN - E
    zpad = jnp.zeros((pad,), jnp.int32)
    dpad = jnp.full((pad,), DUMMY, jnp.int32)
    gidx = jnp.stack([
        jnp.concatenate([i + N_USERS, zpad]),
        jnp.concatenate([u, zpad]),
    ])
    sidx = jnp.stack([
        jnp.concatenate([u, dpad]),
        jnp.concatenate([i, dpad]),
    ])
    pidx = (gidx << 16) | sidx
    xp = lax.bitcast_convert_type(
        x.astype(jnp.bfloat16).reshape(N_USERS + N_ITEMS, DW, 2), jnp.int32)
    sums_e, sums_o, cnt = _sc_accumulate(xp, pidx)
    out = _scale(sums_e, sums_o, cnt.reshape(NC, ROWS, 1))
    n = N_USERS + N_ITEMS
    return out.reshape(n, 2, DW).transpose(0, 2, 1).reshape(n, D)


# speedup vs baseline: 1.0856x; 1.0856x over previous
import dataclasses
import functools

import jax
import jax.numpy as jnp
from jax import lax
from jax.experimental import pallas as pl
from jax.experimental.pallas import tpu as pltpu
from jax.experimental.pallas import tpu_sc as plsc

N_USERS = 5000
N_ITEMS = 5000
D = 256
E = 160000
NC = 2
NT = 16
LANES = 16
SCAN = 1024
NCHUNK = 160
EPAD = SCAN * NCHUNK
ROWS = 5120
RPT = ROWS // NT
ACCR = RPT + 8
GB = 128
PCAP = 2048
DUMMY = ROWS - 1


def _sc_accumulate(x, pidx):
    mesh = plsc.VectorSubcoreMesh(core_axis_name="c", subcore_axis_name="s")
    cp = pltpu.CompilerParams()
    if "needs_layout_passes" in pltpu.CompilerParams.__dataclass_fields__:
        cp = dataclasses.replace(cp, needs_layout_passes=False)

    @functools.partial(
        pl.kernel,
        mesh=mesh,
        compiler_params=cp,
        out_type=(
            jax.ShapeDtypeStruct((NC, ROWS, D), jnp.float32),
            jax.ShapeDtypeStruct((NC * ROWS,), jnp.float32),
        ),
        scratch_types=[
            pltpu.VMEM((ACCR, D), jnp.float32),
            pltpu.VMEM((GB, D), jnp.float32),
            pltpu.VMEM((2, SCAN), jnp.int32),
            pltpu.VMEM((PCAP,), jnp.int32),
            pltpu.VMEM((PCAP,), jnp.int32),
            pltpu.VMEM((ACCR,), jnp.float32),
            pltpu.SMEM((8,), jnp.int32),
            pltpu.SemaphoreType.DMA,
            pltpu.SemaphoreType.DMA,
        ],
    )
    def k(x_hbm, pidx_hbm, sums_hbm, cnt_hbm,
          acc, rows, pc, pg, pd, cnt, cur_ref, semg, semi):
        c = lax.axis_index("c")
        s = lax.axis_index("s")
        lo = s * RPT

        @pl.loop(0, ACCR)
        def _(r):
            for j in range(0, D, LANES):
                acc.at[r, pl.ds(j, LANES)][...] = jnp.zeros(
                    (LANES,), jnp.float32)

        @pl.loop(0, ACCR, step=LANES)
        def _(r):
            cnt.at[pl.ds(r, LANES)][...] = jnp.zeros((LANES,), jnp.float32)

        cur_ref[0] = 0
        ones_f = jnp.ones((LANES,), jnp.float32)

        def drain_full_batches():
            cur = cur_ref[0]
            nb = cur // GB

            @pl.when(nb > 0)
            def _():
                def body(q, carry):
                    base = q * GB
                    pltpu.make_async_copy(
                        x_hbm.at[pg.at[pl.ds(base, GB)]], rows, semg).start()
                    pltpu.make_async_copy(
                        x_hbm.at[pg.at[pl.ds(base, GB)]], rows, semg).wait()

                    @pl.loop(0, GB, step=LANES)
                    def _(e):
                        d_v = pd.at[pl.ds(base + e, LANES)][...]
                        plsc.addupdate_scatter(cnt, [d_v], ones_f)
                        dd = [d_v[li] for li in range(LANES)]

                        def loads(li):
                            return [rows.at[e + li, pl.ds(j, LANES)][...]
                                    for j in range(0, D, LANES)]

                        vals = loads(0)
                        for li in range(LANES):
                            cur_vals = vals
                            if li + 1 < LANES:
                                vals = loads(li + 1)
                            for jq, j in enumerate(range(0, D, LANES)):
                                plsc.addupdate(
                                    acc.at[dd[li], pl.ds(j, LANES)],
                                    cur_vals[jq])

                    return carry

                lax.fori_loop(0, nb, body, 0)

                rbase = nb * GB
                for j in range(0, GB, LANES):
                    pg.at[pl.ds(j, LANES)][...] = (
                        pg.at[pl.ds(rbase + j, LANES)][...])
                    pd.at[pl.ds(j, LANES)][...] = (
                        pd.at[pl.ds(rbase + j, LANES)][...])
                cur_ref[0] = cur - nb * GB

        def idx_copy(chunk, b, start):
            h = pltpu.make_async_copy(
                pidx_hbm.at[c, pl.ds(chunk * SCAN, SCAN)], pc.at[b], semi)
            if start:
                h.start()
            else:
                h.wait()

        def chunk_body(ch, b):
            idx_copy(ch, b, start=False)

            def grp(gi, cur):
                base = gi * LANES
                w_v = pc.at[b, pl.ds(base, LANES)][...]
                g_v = lax.shift_right_logical(w_v, 16)
                d_v = (w_v & 0xFFFF) - lo
                m = plsc.bitcast(d_v, jnp.uint32) < jnp.uint32(RPT)
                plsc.store_compressed(pg.at[pl.ds(cur, LANES)], g_v, mask=m)
                plsc.store_compressed(pd.at[pl.ds(cur, LANES)], d_v, mask=m)
                return cur + plsc.all_reduce_population_count(m)[0]

            cur_ref[0] = lax.fori_loop(0, SCAN // LANES, grp, cur_ref[0])
            idx_copy(ch + 2, b, start=True)
            drain_full_batches()

        idx_copy(0, 0, start=True)
        idx_copy(1, 1, start=True)

        @pl.loop(0, NCHUNK, step=2)
        def _(ch):
            chunk_body(ch, 0)
            chunk_body(ch + 1, 1)

        idx_copy(NCHUNK, 0, start=False)
        idx_copy(NCHUNK + 1, 1, start=False)

        cur = cur_ref[0]
        for j in range(0, GB, LANES):
            pg.at[pl.ds(cur + j, LANES)][...] = jnp.zeros((LANES,), jnp.int32)
            pd.at[pl.ds(cur + j, LANES)][...] = jnp.full(
                (LANES,), RPT, jnp.int32)
        cur_ref[0] = cur + GB
        drain_full_batches()

        pltpu.sync_copy(acc.at[pl.ds(0, RPT)], sums_hbm.at[c, pl.ds(lo, RPT)])
        pltpu.sync_copy(cnt.at[pl.ds(0, RPT)],
                        cnt_hbm.at[pl.ds(c * ROWS + lo, RPT)])

    return k(x, pidx)


_BR = 200


def _scale_body(sums_ref, cnt_ref, out_ref):
    w = 1.0 / jnp.maximum(cnt_ref[0], 1.0)
    out_ref[...] = sums_ref[0] * w


def _scale(sums, cnt):
    nb = N_USERS // _BR
    return pl.pallas_call(
        _scale_body,
        grid=(NC, nb),
        in_specs=[
            pl.BlockSpec((1, _BR, D), lambda c, b: (c, b, 0)),
            pl.BlockSpec((1, _BR, 1), lambda c, b: (c, b, 0)),
        ],
        out_specs=pl.BlockSpec((_BR, D), lambda c, b: (c * nb + b, 0)),
        out_shape=jax.ShapeDtypeStruct((N_USERS + N_ITEMS, D), jnp.float32),
    )(sums, cnt)


def kernel(x, u, i):
    pad = EPAD + 2 * SCAN - E
    zpad = jnp.zeros((pad,), jnp.int32)
    dpad = jnp.full((pad,), DUMMY, jnp.int32)
    gidx = jnp.stack([
        jnp.concatenate([i + N_USERS, zpad]),
        jnp.concatenate([u, zpad]),
    ])
    sidx = jnp.stack([
        jnp.concatenate([u, dpad]),
        jnp.concatenate([i, dpad]),
    ])
    pidx = (gidx << 16) | sidx
    sums, cnt = _sc_accumulate(x, pidx)
    return _scale(sums, cnt.reshape(NC, ROWS, 1))
